# Initial kernel scaffold; baseline (speedup 1.0000x reference)
import jax
import jax.numpy as jnp
from jax.experimental import pallas as pl


def kernel(x, ids, token_table, channel_table):
    raise NotImplementedError("write your pallas kernel here")


# speedup vs baseline: 1.1025x; 1.1025x over previous
import functools

import jax
import jax.numpy as jnp
from jax import lax
from jax.experimental import pallas as pl
from jax.experimental.pallas import tpu as pltpu
from jax.experimental.pallas import tpu_sc as plsc

D = 768
NUM_CHANNELS = 16
BATCH = 16
SEQ = 200
ROWS = BATCH * NUM_CHANNELS * SEQ
LANES = 16
NCORES = 2
NSUB = 16
NW = NCORES * NSUB
ROWS_PER_W = ROWS // NW
BLOCKS_PER_W = ROWS_PER_W // SEQ
CHUNK = 40
NCHUNKS = ROWS_PER_W // CHUNK
CHUNKS_PER_BLOCK = SEQ // CHUNK


def _sc_body(x_hbm, ids_hbm, tok_hbm, ch_hbm, out_hbm,
             idx_v, ids_v, ch_v, buf_v, sem_in, sem_out):
    wid = lax.axis_index("s") * NCORES + lax.axis_index("c")
    row0 = wid * ROWS_PER_W

    pltpu.sync_copy(ids_hbm, ids_v)
    pltpu.async_copy(ch_hbm.at[ids_v], ch_v, sem_in).wait()
    pltpu.sync_copy(x_hbm.at[pl.ds(row0, ROWS_PER_W)], idx_v)

    def chunk_body(t, carry):
        blk = t // CHUNKS_PER_BLOCK
        c = (wid * BLOCKS_PER_W + blk) % NUM_CHANNELS
        pltpu.async_copy(
            tok_hbm.at[idx_v.at[pl.ds(t * CHUNK, CHUNK)]],
            buf_v.at[0], sem_in).wait()
        ch_regs = [ch_v[c, pl.ds(j * LANES, LANES)] for j in range(D // LANES)]

        def row_body(i, _):
            for j in range(D // LANES):
                sl = pl.ds(j * LANES, LANES)
                buf_v[0, i, sl] = buf_v[0, i, sl] + ch_regs[j]
            return 0

        lax.fori_loop(0, CHUNK, row_body, 0)
        pltpu.sync_copy(buf_v.at[0],
                        out_hbm.at[pl.ds(row0 + t * CHUNK, CHUNK)])
        return carry

    lax.fori_loop(0, NCHUNKS, chunk_body, 0)


@jax.jit
def _sc_lookup(xf, ids32, token_table, channel_table):
    mesh = plsc.VectorSubcoreMesh(core_axis_name="c", subcore_axis_name="s")
    f = functools.partial(
        pl.kernel,
        mesh=mesh,
        out_type=jax.ShapeDtypeStruct((ROWS, D), jnp.float32),
        scratch_types=[
            pltpu.VMEM((ROWS_PER_W,), jnp.int32),
            pltpu.VMEM((NUM_CHANNELS,), jnp.int32),
            pltpu.VMEM((NUM_CHANNELS, D), jnp.float32),
            pltpu.VMEM((2, CHUNK, D), jnp.float32),
            pltpu.SemaphoreType.DMA,
            pltpu.SemaphoreType.DMA,
        ],
    )(_sc_body)
    return f(xf, ids32, token_table, channel_table)


def kernel(x, ids, token_table, channel_table):
    xf = x.reshape(ROWS).astype(jnp.int32)
    ids32 = ids.astype(jnp.int32)
    out = _sc_lookup(xf, ids32, token_table, channel_table)
    return out.reshape(BATCH, NUM_CHANNELS, SEQ, D)
